# baseline (speedup 1.0000x reference)
import jax
import jax.numpy as jnp
from jax import lax
from jax.experimental import pallas as pl
from jax.experimental.pallas import tpu as pltpu
from jax.experimental.pallas import tpu_sc as plsc

B, N, C = 64, 577, 768
S0 = 86
SEL_STRIDE = 6
M = 29
OUT_TOKENS = 1 + S0 + 2 * M
NEG = float("-inf")

NW = 32
IDXW = 152
_CH = (80, 72)

_F32 = jnp.float32
_SCORE = lax.Precision.DEFAULT


def _topk29(v):
    col = lax.broadcasted_iota(jnp.int32, (B, N), 1)
    picks = []
    for _ in range(M):
        vmax = jnp.max(v, axis=1, keepdims=True)
        idx = jnp.max(jnp.where(v == vmax, col, -1), axis=1)
        picks.append(idx)
        v = jnp.where(col == idx[:, None], NEG, v)
    return jnp.stack(picks[::-1], axis=1), v


def _p1_body(x_ref, rn_ref, a1_ref, v1_ref, nm_scr):
    b = pl.program_id(0)
    xb = x_ref[0]
    tn = xb * rn_ref[0]
    sel_n = jnp.concatenate(
        [x_ref[0, pl.ds(1 + SEL_STRIDE * s, 1), :]
         * rn_ref[0, pl.ds(1 + SEL_STRIDE * s, 1), :] for s in range(S0)],
        axis=0)
    sc = lax.dot_general(sel_n, tn, (((1,), (1,)), ((), ())),
                         precision=_SCORE, preferred_element_type=_F32)
    nm = jnp.max(sc, axis=0, keepdims=True)
    ci = lax.broadcasted_iota(jnp.int32, (1, N), 1)
    is_sel = (ci >= 1) & (ci <= 1 + SEL_STRIDE * (S0 - 1)) & ((ci - 1) % SEL_STRIDE == 0)
    avail = (ci >= 1) & jnp.logical_not(is_sel)
    nm_scr[pl.ds(b, 1), :] = jnp.where(avail, nm, NEG)

    @pl.when(b == B - 1)
    def _():
        a, v = _topk29(nm_scr[...])
        a1_ref[...] = jnp.concatenate(
            [a, jnp.zeros((B, 32 - M), jnp.int32)], axis=1)
        v1_ref[:, 0, :] = v


def _p1(x, rn):
    return pl.pallas_call(
        _p1_body,
        grid=(B,),
        in_specs=[pl.BlockSpec((1, N, C), lambda b: (b, 0, 0)),
                  pl.BlockSpec((1, N, 1), lambda b: (b, 0, 0))],
        out_specs=[pl.BlockSpec((B, 32), lambda b: (0, 0)),
                   pl.BlockSpec((B, 1, N), lambda b: (0, 0, 0))],
        out_shape=[jax.ShapeDtypeStruct((B, 32), jnp.int32),
                   jax.ShapeDtypeStruct((B, 1, N), _F32)],
        scratch_shapes=[pltpu.VMEM((B, N), _F32)],
    )(x, rn)


def _p3_body(a1_ref, x_ref, rn_ref, v1_ref, a1v_ref, idx_ref, nm_scr):
    b = pl.program_id(0)
    xb = x_ref[0]
    tn = xb * rn_ref[0]
    rows = []
    for s in range(M):
        i = a1_ref[b * 32 + s]
        rows.append(x_ref[0, pl.ds(i, 1), :] * rn_ref[0, pl.ds(i, 1), :])
    a1n = jnp.concatenate(rows, axis=0)
    sc2 = lax.dot_general(a1n, tn, (((1,), (1,)), ((), ())),
                          precision=_SCORE, preferred_element_type=_F32)
    m2 = jnp.max(sc2, axis=0, keepdims=True)
    v1 = v1_ref[0]
    nm_scr[pl.ds(b, 1), :] = jnp.where(v1 > NEG, jnp.maximum(v1, m2), NEG)

    @pl.when(b == B - 1)
    def _():
        a2, _ = _topk29(nm_scr[...])
        selc = 1 + SEL_STRIDE * lax.broadcasted_iota(jnp.int32, (B, S0), 1)
        idx145 = jnp.concatenate(
            [jnp.zeros((B, 1), jnp.int32), selc,
             a1v_ref[:, :M], a2, jnp.zeros((B, IDXW - OUT_TOKENS), jnp.int32)],
            axis=1)
        idx_ref[...] = idx145 + N * lax.broadcasted_iota(jnp.int32, (B, IDXW), 0)


def _p3(a1_flat, x, rn, v1, a1v):
    grid_spec = pltpu.PrefetchScalarGridSpec(
        num_scalar_prefetch=1,
        grid=(B,),
        in_specs=[pl.BlockSpec((1, N, C), lambda b, a1: (b, 0, 0)),
                  pl.BlockSpec((1, N, 1), lambda b, a1: (b, 0, 0)),
                  pl.BlockSpec((1, 1, N), lambda b, a1: (b, 0, 0)),
                  pl.BlockSpec((B, 32), lambda b, a1: (0, 0))],
        out_specs=pl.BlockSpec((B, IDXW), lambda b, a1: (0, 0)),
        scratch_shapes=[pltpu.VMEM((B, N), _F32)],
    )
    return pl.pallas_call(
        _p3_body,
        grid_spec=grid_spec,
        out_shape=jax.ShapeDtypeStruct((B, IDXW), jnp.int32),
    )(a1_flat, x, rn, v1, a1v)


_SCH = 80


def _gather_body(x_hbm, idx_hbm, dj_hbm, d2_hbm, out_hbm,
                 idx_v0, idx_v1, dst_vj, dst_v2, buf0, buf1,
                 gsem0, gsem1, ssem0, ssem1):
    wid = lax.axis_index("s") * 2 + lax.axis_index("c")
    idxs = (idx_v0, idx_v0, idx_v1, idx_v1)
    bis = (0, 0, 1, 1)
    offs = (72, 0, 72, 0)
    bufs = (buf0, buf1, buf0, buf1)
    gsems = (gsem0, gsem1, gsem0, gsem1)
    ssems = (ssem0, ssem1, ssem0, ssem1)
    dhbms = (dj_hbm, d2_hbm, dj_hbm, d2_hbm)
    dsts = (dst_vj, dst_v2, dst_vj, dst_v2)

    pltpu.sync_copy(idx_hbm.at[2 * wid], idx_v0)
    pltpu.sync_copy(idx_hbm.at[2 * wid + 1], idx_v1)

    def gstart(c):
        return pltpu.async_copy(
            x_hbm.at[idxs[c].at[pl.ds(offs[c], _SCH)]], bufs[c], gsems[c])

    h = [gstart(0)]
    sc = []
    for c in range(4):
        bi = 2 * wid + bis[c]
        h[c].wait()
        pltpu.sync_copy(dhbms[c].at[bi], dsts[c])
        if sc:
            sc[-1].wait()
        sc.append(pltpu.async_copy(bufs[c], out_hbm.at[dsts[c]], ssems[c]))
        if c + 1 < 4:
            h.append(gstart(c + 1))
    sc[-1].wait()


def _sc_gather(x_flat, idx2d, dj, d2):
    mesh = plsc.VectorSubcoreMesh(core_axis_name="c", subcore_axis_name="s")
    return pl.kernel(
        _gather_body,
        out_type=jax.ShapeDtypeStruct((B * OUT_TOKENS, C), _F32),
        mesh=mesh,
        scratch_types=[pltpu.VMEM((IDXW,), jnp.int32),
                       pltpu.VMEM((IDXW,), jnp.int32),
                       pltpu.VMEM((_SCH,), jnp.int32),
                       pltpu.VMEM((_SCH,), jnp.int32),
                       pltpu.VMEM((_SCH, C), _F32),
                       pltpu.VMEM((_SCH, C), _F32),
                       pltpu.SemaphoreType.DMA,
                       pltpu.SemaphoreType.DMA,
                       pltpu.SemaphoreType.DMA,
                       pltpu.SemaphoreType.DMA],
    )(x_flat, idx2d, dj, d2)


def kernel(x):
    rn = 1.0 / jnp.linalg.norm(x, axis=-1, keepdims=True)
    a1, v1 = _p1(x, rn)
    idx2d = _p3(a1.reshape(-1), x, rn, v1, a1)
    base = OUT_TOKENS * jnp.arange(B, dtype=jnp.int32)[:, None]
    dj = base + jnp.concatenate(
        [jnp.arange(72, OUT_TOKENS, dtype=jnp.int32),
         jnp.arange(7, dtype=jnp.int32)])[None, :]
    d2 = base + jnp.arange(_SCH, dtype=jnp.int32)[None, :]
    rows = _sc_gather(x.reshape(B * N, C), idx2d, dj, d2)
    return rows.reshape(B, OUT_TOKENS, C)


# speedup vs baseline: 1.1378x; 1.1378x over previous
import jax
import jax.numpy as jnp
from jax import lax
from jax.experimental import pallas as pl
from jax.experimental.pallas import tpu as pltpu
from jax.experimental.pallas import tpu_sc as plsc

B, N, C = 64, 577, 768
S0 = 86
SEL_STRIDE = 6
M = 29
OUT_TOKENS = 1 + S0 + 2 * M
NEG = float("-inf")

NW = 32
IDXW = 152
_CH = (80, 72)

_F32 = jnp.float32
_SCORE = lax.Precision.DEFAULT


def _topk29(v):
    col = lax.broadcasted_iota(jnp.int32, (B, N), 1)
    picks = []
    for _ in range(M):
        vmax = jnp.max(v, axis=1, keepdims=True)
        idx = jnp.max(jnp.where(v == vmax, col, -1), axis=1)
        picks.append(idx)
        v = jnp.where(col == idx[:, None], NEG, v)
    return jnp.stack(picks[::-1], axis=1), v


BB = 2


def _p1_body(x_ref, rn_ref, a1_ref, v1_ref, nm_scr):
    g = pl.program_id(0)
    for t in range(BB):
        xb = x_ref[t]
        tn = xb * rn_ref[t]
        sel_n = jnp.concatenate(
            [x_ref[t, pl.ds(1 + SEL_STRIDE * s, 1), :]
             * rn_ref[t, pl.ds(1 + SEL_STRIDE * s, 1), :] for s in range(S0)],
            axis=0)
        sc = lax.dot_general(sel_n, tn, (((1,), (1,)), ((), ())),
                             precision=_SCORE, preferred_element_type=_F32)
        nm = jnp.max(sc, axis=0, keepdims=True)
        ci = lax.broadcasted_iota(jnp.int32, (1, N), 1)
        is_sel = (ci >= 1) & (ci <= 1 + SEL_STRIDE * (S0 - 1)) & ((ci - 1) % SEL_STRIDE == 0)
        avail = (ci >= 1) & jnp.logical_not(is_sel)
        nm_scr[pl.ds(BB * g + t, 1), :] = jnp.where(avail, nm, NEG)

    @pl.when(g == B // BB - 1)
    def _():
        a, v = _topk29(nm_scr[...])
        a1_ref[...] = jnp.concatenate(
            [a, jnp.zeros((B, 32 - M), jnp.int32)], axis=1)
        v1_ref[:, 0, :] = v


def _p1(x, rn):
    return pl.pallas_call(
        _p1_body,
        grid=(B // BB,),
        in_specs=[pl.BlockSpec((BB, N, C), lambda b: (b, 0, 0)),
                  pl.BlockSpec((BB, N, 1), lambda b: (b, 0, 0))],
        out_specs=[pl.BlockSpec((B, 32), lambda b: (0, 0)),
                   pl.BlockSpec((B, 1, N), lambda b: (0, 0, 0))],
        out_shape=[jax.ShapeDtypeStruct((B, 32), jnp.int32),
                   jax.ShapeDtypeStruct((B, 1, N), _F32)],
        scratch_shapes=[pltpu.VMEM((B, N), _F32)],
    )(x, rn)


def _p3_body(a1_ref, x_ref, rn_ref, v1_ref, a1v_ref, idx_ref, nm_scr):
    g = pl.program_id(0)
    for t in range(BB):
        xb = x_ref[t]
        tn = xb * rn_ref[t]
        rows = []
        for s in range(M):
            i = a1_ref[(BB * g + t) * 32 + s]
            rows.append(x_ref[t, pl.ds(i, 1), :] * rn_ref[t, pl.ds(i, 1), :])
        a1n = jnp.concatenate(rows, axis=0)
        sc2 = lax.dot_general(a1n, tn, (((1,), (1,)), ((), ())),
                              precision=_SCORE, preferred_element_type=_F32)
        m2 = jnp.max(sc2, axis=0, keepdims=True)
        v1 = v1_ref[t]
        nm_scr[pl.ds(BB * g + t, 1), :] = jnp.where(
            v1 > NEG, jnp.maximum(v1, m2), NEG)

    @pl.when(g == B // BB - 1)
    def _():
        a2, _ = _topk29(nm_scr[...])
        selc = 1 + SEL_STRIDE * lax.broadcasted_iota(jnp.int32, (B, S0), 1)
        idx145 = jnp.concatenate(
            [jnp.zeros((B, 1), jnp.int32), selc,
             a1v_ref[:, :M], a2, jnp.zeros((B, IDXW - OUT_TOKENS), jnp.int32)],
            axis=1)
        idx_ref[...] = idx145 + N * lax.broadcasted_iota(jnp.int32, (B, IDXW), 0)


def _p3(a1_flat, x, rn, v1, a1v):
    grid_spec = pltpu.PrefetchScalarGridSpec(
        num_scalar_prefetch=1,
        grid=(B // BB,),
        in_specs=[pl.BlockSpec((BB, N, C), lambda b, a1: (b, 0, 0)),
                  pl.BlockSpec((BB, N, 1), lambda b, a1: (b, 0, 0)),
                  pl.BlockSpec((BB, 1, N), lambda b, a1: (b, 0, 0)),
                  pl.BlockSpec((B, 32), lambda b, a1: (0, 0))],
        out_specs=pl.BlockSpec((B, IDXW), lambda b, a1: (0, 0)),
        scratch_shapes=[pltpu.VMEM((B, N), _F32)],
    )
    return pl.pallas_call(
        _p3_body,
        grid_spec=grid_spec,
        out_shape=jax.ShapeDtypeStruct((B, IDXW), jnp.int32),
    )(a1_flat, x, rn, v1, a1v)


def _gather_body(x_hbm, idx_hbm, out_hbm, idx_v0, idx_v1, buf0, buf1,
                 sem0, sem1):
    wid = lax.axis_index("s") * 2 + lax.axis_index("c")
    pltpu.sync_copy(idx_hbm.at[2 * wid], idx_v0)
    pltpu.sync_copy(idx_hbm.at[2 * wid + 1], idx_v1)
    idxs = (idx_v0, idx_v0, idx_v1, idx_v1)
    bis = (0, 0, 1, 1)
    offs = (0, _CH[0], 0, _CH[0])
    szs = (_CH[0], _CH[1], _CH[0], _CH[1])
    bufs = (buf0, buf1, buf0, buf1)
    sems = (sem0, sem1, sem0, sem1)

    def start(c):
        return pltpu.async_copy(
            x_hbm.at[idxs[c].at[pl.ds(offs[c], szs[c])]],
            bufs[c].at[pl.ds(0, szs[c])], sems[c])

    h = [start(0)]
    for c in range(4):
        h[c].wait()
        if c + 1 < 4:
            h.append(start(c + 1))
        pltpu.sync_copy(bufs[c].at[pl.ds(0, szs[c])],
                        out_hbm.at[2 * wid + bis[c]].at[pl.ds(offs[c], szs[c])])


def _sc_gather(x_flat, idx2d):
    mesh = plsc.VectorSubcoreMesh(core_axis_name="c", subcore_axis_name="s")
    return pl.kernel(
        _gather_body,
        out_type=jax.ShapeDtypeStruct((B, IDXW, C), _F32),
        mesh=mesh,
        scratch_types=[pltpu.VMEM((IDXW,), jnp.int32),
                       pltpu.VMEM((IDXW,), jnp.int32),
                       pltpu.VMEM((_CH[0], C), _F32),
                       pltpu.VMEM((_CH[0], C), _F32),
                       pltpu.SemaphoreType.DMA,
                       pltpu.SemaphoreType.DMA],
    )(x_flat, idx2d)


def kernel(x):
    rn = 1.0 / jnp.linalg.norm(x, axis=-1, keepdims=True)
    a1, v1 = _p1(x, rn)
    idx2d = _p3(a1.reshape(-1), x, rn, v1, a1)
    rows = _sc_gather(x.reshape(B * N, C), idx2d)
    return rows[:, :OUT_TOKENS, :]


# speedup vs baseline: 1.1857x; 1.0421x over previous
import jax
import jax.numpy as jnp
from jax import lax
from jax.experimental import pallas as pl
from jax.experimental.pallas import tpu as pltpu
from jax.experimental.pallas import tpu_sc as plsc

B, N, C = 64, 577, 768
S0 = 86
SEL_STRIDE = 6
M = 29
OUT_TOKENS = 1 + S0 + 2 * M
NEG = float("-inf")

NW = 32
IDXW = 152
_CH = (80, 72)

_F32 = jnp.float32
_SCORE = lax.Precision.DEFAULT


def _topk29(v):
    col = lax.broadcasted_iota(jnp.int32, (B, N), 1)
    picks = []
    for _ in range(M):
        vmax = jnp.max(v, axis=1, keepdims=True)
        idx = jnp.max(jnp.where(v == vmax, col, -1), axis=1)
        picks.append(idx)
        v = jnp.where(col == idx[:, None], NEG, v)
    return jnp.stack(picks[::-1], axis=1), v


BB = 4


def _p1_body(x_ref, rn_ref, a1_ref, v1_ref, nm_scr):
    g = pl.program_id(0)
    for t in range(BB):
        xb = x_ref[t]
        tn = xb * rn_ref[t]
        sel_n = jnp.concatenate(
            [x_ref[t, pl.ds(1 + SEL_STRIDE * s, 1), :]
             * rn_ref[t, pl.ds(1 + SEL_STRIDE * s, 1), :] for s in range(S0)],
            axis=0)
        sc = lax.dot_general(sel_n, tn, (((1,), (1,)), ((), ())),
                             precision=_SCORE, preferred_element_type=_F32)
        nm = jnp.max(sc, axis=0, keepdims=True)
        ci = lax.broadcasted_iota(jnp.int32, (1, N), 1)
        is_sel = (ci >= 1) & (ci <= 1 + SEL_STRIDE * (S0 - 1)) & ((ci - 1) % SEL_STRIDE == 0)
        avail = (ci >= 1) & jnp.logical_not(is_sel)
        nm_scr[pl.ds(BB * g + t, 1), :] = jnp.where(avail, nm, NEG)

    @pl.when(g == B // BB - 1)
    def _():
        a, v = _topk29(nm_scr[...])
        a1_ref[...] = jnp.concatenate(
            [a, jnp.zeros((B, 32 - M), jnp.int32)], axis=1)
        v1_ref[:, 0, :] = v


def _p1(x, rn):
    return pl.pallas_call(
        _p1_body,
        grid=(B // BB,),
        in_specs=[pl.BlockSpec((BB, N, C), lambda b: (b, 0, 0)),
                  pl.BlockSpec((BB, N, 1), lambda b: (b, 0, 0))],
        out_specs=[pl.BlockSpec((B, 32), lambda b: (0, 0)),
                   pl.BlockSpec((B, 1, N), lambda b: (0, 0, 0))],
        out_shape=[jax.ShapeDtypeStruct((B, 32), jnp.int32),
                   jax.ShapeDtypeStruct((B, 1, N), _F32)],
        scratch_shapes=[pltpu.VMEM((B, N), _F32)],
    )(x, rn)


def _p3_body(a1_ref, x_ref, rn_ref, v1_ref, a1v_ref, idx_ref, nm_scr):
    g = pl.program_id(0)
    for t in range(BB):
        xb = x_ref[t]
        tn = xb * rn_ref[t]
        rows = []
        for s in range(M):
            i = a1_ref[(BB * g + t) * 32 + s]
            rows.append(x_ref[t, pl.ds(i, 1), :] * rn_ref[t, pl.ds(i, 1), :])
        a1n = jnp.concatenate(rows, axis=0)
        sc2 = lax.dot_general(a1n, tn, (((1,), (1,)), ((), ())),
                              precision=_SCORE, preferred_element_type=_F32)
        m2 = jnp.max(sc2, axis=0, keepdims=True)
        v1 = v1_ref[t]
        nm_scr[pl.ds(BB * g + t, 1), :] = jnp.where(
            v1 > NEG, jnp.maximum(v1, m2), NEG)

    @pl.when(g == B // BB - 1)
    def _():
        a2, _ = _topk29(nm_scr[...])
        selc = 1 + SEL_STRIDE * lax.broadcasted_iota(jnp.int32, (B, S0), 1)
        idx145 = jnp.concatenate(
            [jnp.zeros((B, 1), jnp.int32), selc,
             a1v_ref[:, :M], a2, jnp.zeros((B, IDXW - OUT_TOKENS), jnp.int32)],
            axis=1)
        idx_ref[...] = idx145 + N * lax.broadcasted_iota(jnp.int32, (B, IDXW), 0)


def _p3(a1_flat, x, rn, v1, a1v):
    grid_spec = pltpu.PrefetchScalarGridSpec(
        num_scalar_prefetch=1,
        grid=(B // BB,),
        in_specs=[pl.BlockSpec((BB, N, C), lambda b, a1: (b, 0, 0)),
                  pl.BlockSpec((BB, N, 1), lambda b, a1: (b, 0, 0)),
                  pl.BlockSpec((BB, 1, N), lambda b, a1: (b, 0, 0)),
                  pl.BlockSpec((B, 32), lambda b, a1: (0, 0))],
        out_specs=pl.BlockSpec((B, IDXW), lambda b, a1: (0, 0)),
        scratch_shapes=[pltpu.VMEM((B, N), _F32)],
    )
    return pl.pallas_call(
        _p3_body,
        grid_spec=grid_spec,
        out_shape=jax.ShapeDtypeStruct((B, IDXW), jnp.int32),
    )(a1_flat, x, rn, v1, a1v)


def _gather_body(x_hbm, idx_hbm, out_hbm, idx_v0, idx_v1, buf0, buf1,
                 sem0, sem1):
    wid = lax.axis_index("s") * 2 + lax.axis_index("c")
    pltpu.sync_copy(idx_hbm.at[2 * wid], idx_v0)
    pltpu.sync_copy(idx_hbm.at[2 * wid + 1], idx_v1)
    idxs = (idx_v0, idx_v0, idx_v1, idx_v1)
    bis = (0, 0, 1, 1)
    offs = (0, _CH[0], 0, _CH[0])
    szs = (_CH[0], _CH[1], _CH[0], _CH[1])
    bufs = (buf0, buf1, buf0, buf1)
    sems = (sem0, sem1, sem0, sem1)

    def start(c):
        return pltpu.async_copy(
            x_hbm.at[idxs[c].at[pl.ds(offs[c], szs[c])]],
            bufs[c].at[pl.ds(0, szs[c])], sems[c])

    h = [start(0)]
    for c in range(4):
        h[c].wait()
        if c + 1 < 4:
            h.append(start(c + 1))
        pltpu.sync_copy(bufs[c].at[pl.ds(0, szs[c])],
                        out_hbm.at[2 * wid + bis[c]].at[pl.ds(offs[c], szs[c])])


def _sc_gather(x_flat, idx2d):
    mesh = plsc.VectorSubcoreMesh(core_axis_name="c", subcore_axis_name="s")
    return pl.kernel(
        _gather_body,
        out_type=jax.ShapeDtypeStruct((B, IDXW, C), _F32),
        mesh=mesh,
        scratch_types=[pltpu.VMEM((IDXW,), jnp.int32),
                       pltpu.VMEM((IDXW,), jnp.int32),
                       pltpu.VMEM((_CH[0], C), _F32),
                       pltpu.VMEM((_CH[0], C), _F32),
                       pltpu.SemaphoreType.DMA,
                       pltpu.SemaphoreType.DMA],
    )(x_flat, idx2d)


def kernel(x):
    rn = 1.0 / jnp.linalg.norm(x, axis=-1, keepdims=True)
    a1, v1 = _p1(x, rn)
    idx2d = _p3(a1.reshape(-1), x, rn, v1, a1)
    rows = _sc_gather(x.reshape(B * N, C), idx2d)
    return rows[:, :OUT_TOKENS, :]
